# baseline (speedup 1.0000x reference)
import jax
import jax.numpy as jnp
from jax.experimental import pallas as pl


_RANK_BLOCK = 8


def _body(ctx_ref, rank_ref, sent_ref, out_ref):
    c = ctx_ref.shape[0]
    k = rank_ref.shape[1]
    out_ref[...] = sent_ref[...]
    out_ref[:, 1 : 1 + c, :] = jnp.broadcast_to(
        ctx_ref[...][None], (out_ref.shape[0], c, out_ref.shape[2])
    )
    out_ref[:, 1 + c : 1 + c + k, :] = rank_ref[...]


def kernel(context_embeds, rank_embeds, sentence_embeds):
    num_ranks, max_tokens, dim = sentence_embeds.shape
    c, _ = context_embeds.shape
    _, k, _ = rank_embeds.shape
    rb = _RANK_BLOCK
    grid = (num_ranks // rb,)
    return pl.pallas_call(
        _body,
        grid=grid,
        in_specs=[
            pl.BlockSpec((c, dim), lambda i: (0, 0)),
            pl.BlockSpec((rb, k, dim), lambda i: (i, 0, 0)),
            pl.BlockSpec((rb, max_tokens, dim), lambda i: (i, 0, 0)),
        ],
        out_specs=pl.BlockSpec((rb, max_tokens, dim), lambda i: (i, 0, 0)),
        out_shape=jax.ShapeDtypeStruct((num_ranks, max_tokens, dim), sentence_embeds.dtype),
    )(context_embeds, rank_embeds, sentence_embeds)
